# Initial kernel scaffold; baseline (speedup 1.0000x reference)
import jax
import jax.numpy as jnp
from jax.experimental import pallas as pl


def kernel(content_features, style_features):
    raise NotImplementedError("write your pallas kernel here")


# speedup vs baseline: 2.6962x; 2.6962x over previous
import numpy as np
import jax
import jax.numpy as jnp
from jax.experimental import pallas as pl

P = 3
H = W = 56
L = H * W
C = 96
D = C * P * P
LC_TILE = 112
N_TILES = L // LC_TILE

_HIGH = jax.lax.Precision.HIGHEST


def _recip_divisor_np():
    div = np.zeros((H, W), dtype=np.float32)
    ones = np.ones((H, W), dtype=np.float32)
    for dy in (-1, 0, 1):
        for dx in (-1, 0, 1):
            ys, ye = max(0, dy), H + min(0, dy)
            xs, xe = max(0, dx), W + min(0, dx)
            div[ys:ye, xs:xe] += ones[max(0, -dy):H + min(0, -dy),
                                      max(0, -dx):W + min(0, -dx)]
    return (np.float32(1.0) / (div + np.float32(1e-8)))


_RECIP_DIV = _recip_divisor_np()


def _match_body(c_ref, sn_ref, maxn_ref, out_ref):
    c_tile = c_ref[...]
    sn = sn_ref[...]
    maxn = maxn_ref[...]
    rn = jnp.sqrt(jnp.sum(c_tile * c_tile, axis=1, keepdims=True))
    cn = c_tile / jnp.maximum(rn, 1e-12)
    scores = jax.lax.dot_general(
        cn, sn, (((1,), (0,)), ((), ())))
    iota = jax.lax.broadcasted_iota(jnp.int32, (LC_TILE, L), 1)
    m = jnp.max(scores, axis=1, keepdims=True)
    best = jnp.min(jnp.where(scores == m, iota, L), axis=1)
    onehot = (iota == best[:, None]).astype(jnp.float32)
    matched_t = jax.lax.dot_general(
        sn, onehot, (((1,), (1,)), ((), ())), precision=_HIGH)
    norm_t = jax.lax.dot_general(
        maxn, onehot, (((1,), (1,)), ((), ())), precision=_HIGH)
    out_ref[0] = matched_t * norm_t


def _fold_body(cols_ref, recip_ref, out_ref):
    acc = None
    for k in range(P * P):
        dy, dx = k // P - 1, k % P - 1
        img = cols_ref[k]
        crop = img[:, max(0, -dy):H + min(0, -dy),
                   max(0, -dx):W + min(0, -dx)]
        ty, tx = max(0, dy), max(0, dx)
        placed = jnp.pad(crop, ((0, 0),
                                (ty, H - ty - crop.shape[1]),
                                (tx, W - tx - crop.shape[2])))
        acc = placed if acc is None else acc + placed
    out_ref[...] = acc * recip_ref[...][None, :, :]


def _patches_t(x):
    xp = jnp.pad(x, ((0, 0), (1, 1), (1, 1)))
    shifted = jnp.stack([xp[:, i:i + H, j:j + W]
                         for i in range(P) for j in range(P)], axis=0)
    return shifted.reshape(P * P * C, L)


def kernel(content_features, style_features):
    ct = _patches_t(content_features[0])
    st = _patches_t(style_features[0])
    cflat = ct.T
    maxn = jnp.maximum(jnp.sqrt(jnp.sum(st * st, axis=0, keepdims=True)),
                       1e-12)
    sn = st / maxn

    matched_t = pl.pallas_call(
        _match_body,
        grid=(N_TILES,),
        in_specs=[
            pl.BlockSpec((LC_TILE, D), lambda i: (i, 0)),
            pl.BlockSpec((D, L), lambda i: (0, 0)),
            pl.BlockSpec((1, L), lambda i: (0, 0)),
        ],
        out_specs=pl.BlockSpec((1, D, LC_TILE), lambda i: (i, 0, 0)),
        out_shape=jax.ShapeDtypeStruct((N_TILES, D, LC_TILE), jnp.float32),
    )(cflat, sn, maxn)

    cols = jnp.transpose(matched_t, (1, 0, 2)).reshape(D, L)
    cols = cols.reshape(P * P, C, H, W)

    out = pl.pallas_call(
        _fold_body,
        in_specs=[
            pl.BlockSpec((P * P, C, H, W), lambda: (0, 0, 0, 0)),
            pl.BlockSpec((H, W), lambda: (0, 0)),
        ],
        out_specs=pl.BlockSpec((C, H, W), lambda: (0, 0, 0)),
        out_shape=jax.ShapeDtypeStruct((C, H, W), jnp.float32),
    )(cols, jnp.asarray(_RECIP_DIV))
    return out[None]


# speedup vs baseline: 6.1881x; 2.2951x over previous
import functools

import numpy as np
import jax
import jax.numpy as jnp
from jax import lax
from jax.experimental import pallas as pl
from jax.experimental.pallas import tpu as pltpu
from jax.experimental.pallas import tpu_sc as plsc

P = 3
H = W = 56
L = H * W
C = 96
D = C * P * P
LC_TILE = 512
L_PAD = 3584
N_TILES = L_PAD // LC_TILE

HP = H + 2
NPIX = HP * HP

NSC, NSUB = 2, 16
NW = NSC * NSUB
R = L * P * P
CHUNK = 896
R_PAD = NW * CHUNK
IDX_W = 128
NCH = CHUNK // IDX_W

JUNK_ROW = L
ACC_ROWS = 3328
ACC_SLICE = ACC_ROWS // NSUB
CW = 128


def _recip_divisor_np():
    div = np.zeros((H, W), dtype=np.float32)
    for dy in (-1, 0, 1):
        for dx in (-1, 0, 1):
            div[max(0, dy):H + min(0, dy), max(0, dx):W + min(0, dx)] += 1.0
    return np.float32(1.0) / (div + np.float32(1e-8))


_RECIP_PIX = _recip_divisor_np().reshape(L, 1)


def _scatter_idx_np():
    l = np.arange(L)[:, None]
    k = np.arange(P * P)[None, :]
    y, x = l // W, l % W
    ty, tx = y + k // P - 1, x + k % P - 1
    idx = np.where((ty >= 0) & (ty < H) & (tx >= 0) & (tx < W),
                   ty * W + tx, JUNK_ROW).astype(np.int32)
    flat = np.full((R_PAD,), JUNK_ROW, dtype=np.int32)
    flat[:R] = idx.reshape(-1)
    return flat.reshape(NW, NCH, IDX_W)


_SCATTER_IDX = _scatter_idx_np()


def _match_body(ct_ref, sn_ref, gidx_ref):
    ct = ct_ref[...]
    sn = sn_ref[...]
    rn = jnp.sqrt(jnp.sum(ct * ct, axis=0, keepdims=True))
    cn = ct / jnp.maximum(rn, 1e-12)
    scores = jax.lax.dot_general(
        cn, sn, (((0,), (0,)), ((), ())))
    iota = lax.broadcasted_iota(jnp.int32, (LC_TILE, L), 1)
    m = jnp.max(scores, axis=1, keepdims=True)
    best = jnp.min(jnp.where(scores == m, iota, L), axis=1)
    yb = jnp.right_shift(best * 18725, 20)
    base = best + 2 * yb
    ki = lax.broadcasted_iota(jnp.int32, (LC_TILE, P * P), 1)
    ti = jnp.right_shift(ki * 21846, 16)
    off = ti * (HP - P) + ki
    gidx_ref[0] = base[:, None] + off


def _combine_body(parts_ref, recip_ref, out_ref):
    p = parts_ref[0] + parts_ref[1]
    img = p[:L, :C] * recip_ref[...]
    out_ref[...] = img.T


def _sc_body(gidx_hbm, sidx_hbm, table_hbm, zeros_hbm, out_hbm,
             gidx_v, sidx_v, rows_v, acc_sh, sems):
    cid = lax.axis_index("c")
    sid = lax.axis_index("s")
    wid = cid * NSUB + sid
    pltpu.sync_copy(gidx_hbm.at[wid], gidx_v)
    pltpu.sync_copy(sidx_hbm.at[wid], sidx_v)
    pltpu.sync_copy(zeros_hbm.at[pl.ds(sid * ACC_SLICE, ACC_SLICE)],
                    acc_sh.at[pl.ds(sid * ACC_SLICE, ACC_SLICE)])
    plsc.subcore_barrier()
    descs = [None, None]
    descs[0] = pltpu.async_copy(table_hbm.at[gidx_v.at[0]],
                                rows_v.at[0], sems.at[0])
    for j in range(NCH):
        b = j & 1
        nb = (j + 1) & 1
        descs[b].wait()
        if j + 1 < NCH:
            descs[nb] = pltpu.async_copy(table_hbm.at[gidx_v.at[j + 1]],
                                         rows_v.at[nb], sems.at[nb])
        pltpu.sync_copy(rows_v.at[b], acc_sh.at[sidx_v.at[j]], add=True)
    plsc.subcore_barrier()
    pltpu.sync_copy(acc_sh.at[pl.ds(sid * ACC_SLICE, ACC_SLICE)],
                    out_hbm.at[cid].at[pl.ds(sid * ACC_SLICE, ACC_SLICE)])


@functools.cache
def _sc_gather_fold():
    return pl.kernel(
        _sc_body,
        out_type=jax.ShapeDtypeStruct((NSC, ACC_ROWS, CW), jnp.float32),
        mesh=plsc.VectorSubcoreMesh(core_axis_name="c", subcore_axis_name="s",
                                    num_cores=NSC, num_subcores=NSUB),
        scratch_types=[
            pltpu.VMEM((NCH, IDX_W), jnp.int32),
            pltpu.VMEM((NCH, IDX_W), jnp.int32),
            pltpu.VMEM((2, IDX_W, CW), jnp.float32),
            pltpu.VMEM_SHARED((ACC_ROWS, CW), jnp.float32),
            pltpu.SemaphoreType.DMA((2,)),
        ],
    )


def _patches_t(x):
    xp = jnp.pad(x, ((0, 0), (1, 1), (1, 1)))
    shifted = jnp.stack([xp[:, i:i + H, j:j + W]
                         for i in range(P) for j in range(P)], axis=0)
    return shifted.reshape(P * P * C, L)


def kernel(content_features, style_features):
    ct = _patches_t(content_features[0])
    st = _patches_t(style_features[0])
    maxn = jnp.maximum(jnp.sqrt(jnp.sum(st * st, axis=0, keepdims=True)),
                       1e-12)
    sn = st / maxn

    ct_pad = jnp.pad(ct, ((0, 0), (0, L_PAD - L)))
    gidx = pl.pallas_call(
        _match_body,
        grid=(N_TILES,),
        in_specs=[
            pl.BlockSpec((D, LC_TILE), lambda i: (0, i)),
            pl.BlockSpec((D, L), lambda i: (0, 0)),
        ],
        out_specs=pl.BlockSpec((1, LC_TILE, P * P), lambda i: (i, 0, 0)),
        out_shape=jax.ShapeDtypeStruct((N_TILES, LC_TILE, P * P), jnp.int32),
    )(ct_pad, sn)

    gidx_valid = gidx.reshape(L_PAD, P * P)[:L].reshape(R)
    gidx_flat = jnp.full((R_PAD,), 0, dtype=jnp.int32)
    gidx_flat = lax.dynamic_update_slice(gidx_flat, gidx_valid, (0,))
    gidx_w = gidx_flat.reshape(NW, NCH, IDX_W)

    s_hwc = jnp.transpose(style_features[0], (1, 2, 0))
    table = jnp.pad(s_hwc, ((1, 1), (1, 1), (0, CW - C))).reshape(NPIX, CW)

    parts = _sc_gather_fold()(
        gidx_w, jnp.asarray(_SCATTER_IDX), table,
        jnp.zeros((ACC_ROWS, CW), jnp.float32))

    out = pl.pallas_call(
        _combine_body,
        in_specs=[
            pl.BlockSpec((NSC, ACC_ROWS, CW), lambda: (0, 0, 0)),
            pl.BlockSpec((L, 1), lambda: (0, 0)),
        ],
        out_specs=pl.BlockSpec((C, L), lambda: (0, 0)),
        out_shape=jax.ShapeDtypeStruct((C, L), jnp.float32),
    )(parts, jnp.asarray(_RECIP_PIX))
    return out.reshape(1, C, H, W)


# speedup vs baseline: 6.6701x; 1.0779x over previous
import functools

import numpy as np
import jax
import jax.numpy as jnp
from jax import lax
from jax.experimental import pallas as pl
from jax.experimental.pallas import tpu as pltpu
from jax.experimental.pallas import tpu_sc as plsc

P = 3
H = W = 56
L = H * W
C = 96
D = C * P * P
LC_TILE = 512
L_PAD = 3584
N_TILES = L_PAD // LC_TILE

HP = H + 2
NPIX = HP * HP

NSC, NSUB = 2, 16
NW = NSC * NSUB
R = L * P * P
CHUNK = 896
R_PAD = NW * CHUNK
IDX_W = 128
NCH = CHUNK // IDX_W
NBUF = 4

JUNK_ROW = L
ACC_ROWS = 3328
ACC_SLICE = ACC_ROWS // NSUB
CW = 128


def _recip_divisor_np():
    div = np.zeros((H, W), dtype=np.float32)
    for dy in (-1, 0, 1):
        for dx in (-1, 0, 1):
            div[max(0, dy):H + min(0, dy), max(0, dx):W + min(0, dx)] += 1.0
    return np.float32(1.0) / (div + np.float32(1e-8))


_RECIP_PIX = _recip_divisor_np().reshape(L, 1)


def _scatter_idx_np():
    l = np.arange(L)[:, None]
    k = np.arange(P * P)[None, :]
    y, x = l // W, l % W
    ty, tx = y + k // P - 1, x + k % P - 1
    idx = np.where((ty >= 0) & (ty < H) & (tx >= 0) & (tx < W),
                   ty * W + tx, JUNK_ROW).astype(np.int32)
    flat = np.full((R_PAD,), JUNK_ROW, dtype=np.int32)
    flat[:R] = idx.reshape(-1)
    return flat.reshape(NW, NCH, IDX_W)


_SCATTER_IDX = _scatter_idx_np()


def _match_body(ct_ref, sn_ref, gidx_ref):
    ct = ct_ref[...]
    sn = sn_ref[...]
    rn = jnp.sqrt(jnp.sum(ct * ct, axis=0, keepdims=True))
    cn = ct / jnp.maximum(rn, 1e-12)
    scores = jax.lax.dot_general(
        cn, sn, (((0,), (0,)), ((), ())))
    iota = lax.broadcasted_iota(jnp.int32, (LC_TILE, L), 1)
    m = jnp.max(scores, axis=1, keepdims=True)
    best = jnp.min(jnp.where(scores == m, iota, L), axis=1)
    yb = jnp.right_shift(best * 18725, 20)
    base = best + 2 * yb
    ki = lax.broadcasted_iota(jnp.int32, (LC_TILE, P * P), 1)
    ti = jnp.right_shift(ki * 21846, 16)
    off = ti * (HP - P) + ki
    gidx_ref[0] = base[:, None] + off


def _combine_body(parts_ref, recip_ref, out_ref):
    p = parts_ref[0] + parts_ref[1]
    img = p[:L, :C] * recip_ref[...]
    out_ref[...] = img.T


def _sc_body(gidx_hbm, sidx_hbm, table_hbm, zeros_hbm, out_hbm,
             gidx_v, sidx_v, rows_v, acc_sh, sems):
    cid = lax.axis_index("c")
    sid = lax.axis_index("s")
    wid = cid * NSUB + sid
    pltpu.sync_copy(gidx_hbm.at[wid], gidx_v)
    pltpu.sync_copy(sidx_hbm.at[wid], sidx_v)
    pltpu.sync_copy(zeros_hbm.at[pl.ds(sid * ACC_SLICE, ACC_SLICE)],
                    acc_sh.at[pl.ds(sid * ACC_SLICE, ACC_SLICE)])
    plsc.subcore_barrier()
    descs = [None] * NBUF
    for j in range(min(NBUF - 1, NCH)):
        descs[j] = pltpu.async_copy(table_hbm.at[gidx_v.at[j]],
                                    rows_v.at[j], sems.at[j])
    for j in range(NCH):
        b = j % NBUF
        descs[b].wait()
        nj = j + NBUF - 1
        if nj < NCH:
            descs[nj % NBUF] = pltpu.async_copy(
                table_hbm.at[gidx_v.at[nj]], rows_v.at[nj % NBUF],
                sems.at[nj % NBUF])
        pltpu.sync_copy(rows_v.at[b], acc_sh.at[sidx_v.at[j]], add=True)
    plsc.subcore_barrier()
    pltpu.sync_copy(acc_sh.at[pl.ds(sid * ACC_SLICE, ACC_SLICE)],
                    out_hbm.at[cid].at[pl.ds(sid * ACC_SLICE, ACC_SLICE)])


@functools.cache
def _sc_gather_fold():
    return pl.kernel(
        _sc_body,
        out_type=jax.ShapeDtypeStruct((NSC, ACC_ROWS, CW), jnp.float32),
        mesh=plsc.VectorSubcoreMesh(core_axis_name="c", subcore_axis_name="s",
                                    num_cores=NSC, num_subcores=NSUB),
        scratch_types=[
            pltpu.VMEM((NCH, IDX_W), jnp.int32),
            pltpu.VMEM((NCH, IDX_W), jnp.int32),
            pltpu.VMEM((NBUF, IDX_W, CW), jnp.float32),
            pltpu.VMEM_SHARED((ACC_ROWS, CW), jnp.float32),
            pltpu.SemaphoreType.DMA((NBUF,)),
        ],
    )


def _patches_t(x, h_out=H):
    xp = jnp.pad(x, ((0, 0), (1, 1 + h_out - H), (1, 1)))
    shifted = jnp.stack([xp[:, i:i + h_out, j:j + W]
                         for i in range(P) for j in range(P)], axis=0)
    return shifted.reshape(P * P * C, h_out * W)


def kernel(content_features, style_features):
    ct_pad = _patches_t(content_features[0], h_out=L_PAD // W)
    sp = jnp.pad(style_features[0], ((0, 0), (1, 1), (1, 1)))
    ssq = jnp.sum(sp * sp, axis=0)
    n2 = sum(ssq[i:i + H, j:j + W]
             for i in range(P) for j in range(P)).reshape(1, L)
    maxn = jnp.maximum(jnp.sqrt(n2), 1e-12)
    sn = _patches_t(style_features[0]) / maxn
    gidx = pl.pallas_call(
        _match_body,
        grid=(N_TILES,),
        in_specs=[
            pl.BlockSpec((D, LC_TILE), lambda i: (0, i)),
            pl.BlockSpec((D, L), lambda i: (0, 0)),
        ],
        out_specs=pl.BlockSpec((1, LC_TILE, P * P), lambda i: (i, 0, 0)),
        out_shape=jax.ShapeDtypeStruct((N_TILES, LC_TILE, P * P), jnp.int32),
    )(ct_pad, sn)

    gidx_valid = gidx.reshape(L_PAD, P * P)[:L].reshape(R)
    gidx_flat = jnp.full((R_PAD,), 0, dtype=jnp.int32)
    gidx_flat = lax.dynamic_update_slice(gidx_flat, gidx_valid, (0,))
    gidx_w = gidx_flat.reshape(NW, NCH, IDX_W)

    s_hwc = jnp.transpose(style_features[0], (1, 2, 0))
    table = jnp.pad(s_hwc, ((1, 1), (1, 1), (0, CW - C))).reshape(NPIX, CW)

    parts = _sc_gather_fold()(
        gidx_w, jnp.asarray(_SCATTER_IDX), table,
        jnp.zeros((ACC_ROWS, CW), jnp.float32))

    out = pl.pallas_call(
        _combine_body,
        in_specs=[
            pl.BlockSpec((NSC, ACC_ROWS, CW), lambda: (0, 0, 0)),
            pl.BlockSpec((L, 1), lambda: (0, 0)),
        ],
        out_specs=pl.BlockSpec((C, L), lambda: (0, 0)),
        out_shape=jax.ShapeDtypeStruct((C, L), jnp.float32),
    )(parts, jnp.asarray(_RECIP_PIX))
    return out.reshape(1, C, H, W)


# speedup vs baseline: 6.7217x; 1.0077x over previous
import functools

import numpy as np
import jax
import jax.numpy as jnp
from jax import lax
from jax.experimental import pallas as pl
from jax.experimental.pallas import tpu as pltpu
from jax.experimental.pallas import tpu_sc as plsc

P = 3
H = W = 56
L = H * W
C = 96
D = C * P * P
LC_TILE = 512
L_PAD = 3584
N_TILES = L_PAD // LC_TILE

HP = H + 2
NPIX = HP * HP

NSC, NSUB = 2, 16
NW = NSC * NSUB
R = L * P * P
CHUNK = 896
R_PAD = NW * CHUNK
IDX_W = 128
NCH = CHUNK // IDX_W
NBUF = 4

JUNK_ROW = L
ACC_ROWS = 3200
ACC_SLICE = ACC_ROWS // NSUB
CW = 128


def _recip_divisor_np():
    div = np.zeros((H, W), dtype=np.float32)
    for dy in (-1, 0, 1):
        for dx in (-1, 0, 1):
            div[max(0, dy):H + min(0, dy), max(0, dx):W + min(0, dx)] += 1.0
    return np.float32(1.0) / (div + np.float32(1e-8))


_RECIP_PIX = _recip_divisor_np().reshape(L, 1)


def _scatter_idx_np():
    l = np.arange(L)[:, None]
    k = np.arange(P * P)[None, :]
    y, x = l // W, l % W
    ty, tx = y + k // P - 1, x + k % P - 1
    idx = np.where((ty >= 0) & (ty < H) & (tx >= 0) & (tx < W),
                   ty * W + tx, JUNK_ROW).astype(np.int32)
    flat = np.full((R_PAD,), JUNK_ROW, dtype=np.int32)
    flat[:R] = idx.reshape(-1)
    return flat.reshape(NW, NCH, IDX_W)


_SCATTER_IDX = _scatter_idx_np()


def _match_body(ct_ref, sn_ref, gidx_ref):
    ct = ct_ref[...]
    sn = sn_ref[...]
    rn = jnp.sqrt(jnp.sum(ct * ct, axis=0, keepdims=True))
    cn = ct / jnp.maximum(rn, 1e-12)
    scores = jax.lax.dot_general(
        cn, sn, (((0,), (0,)), ((), ())))
    iota = lax.broadcasted_iota(jnp.int32, (LC_TILE, L), 1)
    m = jnp.max(scores, axis=1, keepdims=True)
    best = jnp.min(jnp.where(scores == m, iota, L), axis=1)
    yb = jnp.right_shift(best * 18725, 20)
    base = best + 2 * yb
    ki = lax.broadcasted_iota(jnp.int32, (LC_TILE, P * P), 1)
    ti = jnp.right_shift(ki * 21846, 16)
    off = ti * (HP - P) + ki
    gidx_ref[0] = base[:, None] + off


def _combine_body(parts_ref, recip_ref, out_ref):
    p = parts_ref[0] + parts_ref[1]
    img = p[:L, :C] * recip_ref[...]
    out_ref[...] = img.T


def _sc_body(gidx_hbm, sidx_hbm, table_hbm, zeros_hbm, out_hbm,
             gidx_v, sidx_v, rows_v, acc_sh, sems, zsem):
    cid = lax.axis_index("c")
    sid = lax.axis_index("s")
    wid = cid * NSUB + sid
    pltpu.sync_copy(gidx_hbm.at[wid], gidx_v)
    pltpu.sync_copy(sidx_hbm.at[wid], sidx_v)
    zdesc = pltpu.async_copy(zeros_hbm.at[pl.ds(sid * ACC_SLICE, ACC_SLICE)],
                             acc_sh.at[pl.ds(sid * ACC_SLICE, ACC_SLICE)],
                             zsem)
    descs = [None] * NBUF
    for j in range(min(NBUF - 1, NCH)):
        descs[j] = pltpu.async_copy(table_hbm.at[gidx_v.at[j]],
                                    rows_v.at[j], sems.at[j])
    zdesc.wait()
    plsc.subcore_barrier()
    for j in range(NCH):
        b = j % NBUF
        descs[b].wait()
        nj = j + NBUF - 1
        if nj < NCH:
            descs[nj % NBUF] = pltpu.async_copy(
                table_hbm.at[gidx_v.at[nj]], rows_v.at[nj % NBUF],
                sems.at[nj % NBUF])
        pltpu.sync_copy(rows_v.at[b], acc_sh.at[sidx_v.at[j]], add=True)
    plsc.subcore_barrier()
    pltpu.sync_copy(acc_sh.at[pl.ds(sid * ACC_SLICE, ACC_SLICE)],
                    out_hbm.at[cid].at[pl.ds(sid * ACC_SLICE, ACC_SLICE)])


@functools.cache
def _sc_gather_fold():
    return pl.kernel(
        _sc_body,
        out_type=jax.ShapeDtypeStruct((NSC, ACC_ROWS, CW), jnp.float32),
        mesh=plsc.VectorSubcoreMesh(core_axis_name="c", subcore_axis_name="s",
                                    num_cores=NSC, num_subcores=NSUB),
        scratch_types=[
            pltpu.VMEM((NCH, IDX_W), jnp.int32),
            pltpu.VMEM((NCH, IDX_W), jnp.int32),
            pltpu.VMEM((NBUF, IDX_W, CW), jnp.float32),
            pltpu.VMEM_SHARED((ACC_ROWS, CW), jnp.float32),
            pltpu.SemaphoreType.DMA((NBUF,)),
            pltpu.SemaphoreType.DMA,
        ],
    )


def _patches_t(x, h_out=H):
    xp = jnp.pad(x, ((0, 0), (1, 1 + h_out - H), (1, 1)))
    shifted = jnp.stack([xp[:, i:i + h_out, j:j + W]
                         for i in range(P) for j in range(P)], axis=0)
    return shifted.reshape(P * P * C, h_out * W)


def kernel(content_features, style_features):
    ct_pad = _patches_t(content_features[0], h_out=L_PAD // W)
    sp = jnp.pad(style_features[0], ((0, 0), (1, 1), (1, 1)))
    ssq = jnp.sum(sp * sp, axis=0)
    n2 = sum(ssq[i:i + H, j:j + W]
             for i in range(P) for j in range(P)).reshape(1, L)
    maxn = jnp.maximum(jnp.sqrt(n2), 1e-12)
    sn = _patches_t(style_features[0]) / maxn
    gidx = pl.pallas_call(
        _match_body,
        grid=(N_TILES,),
        in_specs=[
            pl.BlockSpec((D, LC_TILE), lambda i: (0, i)),
            pl.BlockSpec((D, L), lambda i: (0, 0)),
        ],
        out_specs=pl.BlockSpec((1, LC_TILE, P * P), lambda i: (i, 0, 0)),
        out_shape=jax.ShapeDtypeStruct((N_TILES, LC_TILE, P * P), jnp.int32),
    )(ct_pad, sn)

    gidx_valid = gidx.reshape(L_PAD, P * P)[:L].reshape(R)
    gidx_flat = jnp.full((R_PAD,), 0, dtype=jnp.int32)
    gidx_flat = lax.dynamic_update_slice(gidx_flat, gidx_valid, (0,))
    gidx_w = gidx_flat.reshape(NW, NCH, IDX_W)

    s_hwc = jnp.transpose(style_features[0], (1, 2, 0))
    table = jnp.pad(s_hwc, ((1, 1), (1, 1), (0, CW - C))).reshape(NPIX, CW)

    parts = _sc_gather_fold()(
        gidx_w, jnp.asarray(_SCATTER_IDX), table,
        jnp.zeros((ACC_ROWS, CW), jnp.float32))

    out = pl.pallas_call(
        _combine_body,
        in_specs=[
            pl.BlockSpec((NSC, ACC_ROWS, CW), lambda: (0, 0, 0)),
            pl.BlockSpec((L, 1), lambda: (0, 0)),
        ],
        out_specs=pl.BlockSpec((C, L), lambda: (0, 0)),
        out_shape=jax.ShapeDtypeStruct((C, L), jnp.float32),
    )(parts, jnp.asarray(_RECIP_PIX))
    return out.reshape(1, C, H, W)
